# baseline (speedup 1.0000x reference)
import functools

import jax
import jax.numpy as jnp
from jax import lax
from jax.experimental import pallas as pl
from jax.experimental.pallas import tpu as pltpu
from jax.experimental.pallas import tpu_sc as plsc

VOCAB = 1000
B = 1024
T = 20
NC, NS = 2, 16
NW = NC * NS
B_PER_W = B // NW
BPC = 2
CHUNK = BPC * T
N_CHUNKS = B_PER_W // BPC

_mesh = plsc.VectorSubcoreMesh(core_axis_name="c", subcore_axis_name="s")


@functools.partial(
    pl.kernel,
    out_type=jax.ShapeDtypeStruct((B, T, VOCAB), jnp.float32),
    mesh=_mesh,
    compiler_params=pltpu.CompilerParams(use_tc_tiling_on_sc=False),
    scratch_types=[
        pltpu.VMEM((B_PER_W * T,), jnp.int32),
        pltpu.VMEM((CHUNK, VOCAB), jnp.float32),
        pltpu.VMEM((CHUNK, VOCAB), jnp.float32),
        pltpu.VMEM((CHUNK, VOCAB), jnp.float32),
        pltpu.SemaphoreType.DMA,
        pltpu.SemaphoreType.DMA,
        pltpu.SemaphoreType.DMA,
        pltpu.SemaphoreType.DMA,
        pltpu.SemaphoreType.DMA,
        pltpu.SemaphoreType.DMA,
    ],
)
def _gather_rows(emb_hbm, idx_hbm, out_hbm, idx_v,
                 buf0, buf1, buf2, g0, g1, g2, s0, s1, s2):
    wid = lax.axis_index("s") * NC + lax.axis_index("c")
    base_b = wid * B_PER_W
    pltpu.sync_copy(idx_hbm.at[pl.ds(base_b * T, B_PER_W * T)], idx_v)

    bufs = (buf0, buf1, buf2)
    gsems = (g0, g1, g2)
    ssems = (s0, s1, s2)

    def gather(j):
        return pltpu.make_async_copy(
            emb_hbm.at[idx_v.at[pl.ds(j * CHUNK, CHUNK)]],
            bufs[j % 3],
            gsems[j % 3],
        )

    def scatter(j):
        buf = bufs[j % 3]
        a = pltpu.make_async_copy(
            buf.at[pl.ds(0, T)], out_hbm.at[base_b + BPC * j], ssems[j % 3])
        b = pltpu.make_async_copy(
            buf.at[pl.ds(T, T)], out_hbm.at[base_b + BPC * j + 1],
            ssems[j % 3])
        return a, b

    gathers = [gather(0), gather(1)]
    gathers[0].start()
    gathers[1].start()
    scatters = []
    for j in range(N_CHUNKS):
        if j + 2 < N_CHUNKS:
            if j >= 1:
                for cp in scatters[j - 1]:
                    cp.wait()
            g = gather(j + 2)
            g.start()
            gathers.append(g)
        gathers[j].wait()
        a, b = scatter(j)
        a.start()
        b.start()
        scatters.append((a, b))
    for j in (N_CHUNKS - 3, N_CHUNKS - 2, N_CHUNKS - 1):
        for cp in scatters[j]:
            cp.wait()


def kernel(xb, emb):
    idx = xb.reshape(-1)
    return _gather_rows(emb, idx)


# speedup vs baseline: 1.0018x; 1.0018x over previous
import functools

import jax
import jax.numpy as jnp
from jax import lax
from jax.experimental import pallas as pl
from jax.experimental.pallas import tpu as pltpu
from jax.experimental.pallas import tpu_sc as plsc

VOCAB = 1000
B = 1024
T = 20
NC, NS = 2, 16
NW = NC * NS
B_PER_W = B // NW
BPC = 2
CHUNK = BPC * T
N_CHUNKS = B_PER_W // BPC

_mesh = plsc.VectorSubcoreMesh(core_axis_name="c", subcore_axis_name="s")


@functools.partial(
    pl.kernel,
    out_type=jax.ShapeDtypeStruct((B, T, VOCAB), jnp.float32),
    mesh=_mesh,
    compiler_params=pltpu.CompilerParams(use_tc_tiling_on_sc=False),
    scratch_types=[
        pltpu.VMEM((B_PER_W * T,), jnp.int32),
        pltpu.VMEM((CHUNK, 1024), jnp.float32),
        pltpu.VMEM((CHUNK, 1024), jnp.float32),
        pltpu.VMEM((CHUNK, 1024), jnp.float32),
        pltpu.SemaphoreType.DMA,
        pltpu.SemaphoreType.DMA,
        pltpu.SemaphoreType.DMA,
        pltpu.SemaphoreType.DMA,
        pltpu.SemaphoreType.DMA,
        pltpu.SemaphoreType.DMA,
    ],
)
def _gather_rows(emb_hbm, idx_hbm, out_hbm, idx_v,
                 buf0, buf1, buf2, g0, g1, g2, s0, s1, s2):
    wid = lax.axis_index("s") * NC + lax.axis_index("c")
    base_b = wid * B_PER_W
    pltpu.sync_copy(idx_hbm.at[pl.ds(base_b * T, B_PER_W * T)], idx_v)

    bufs = (buf0, buf1, buf2)
    gsems = (g0, g1, g2)
    ssems = (s0, s1, s2)

    def gather(j):
        return pltpu.make_async_copy(
            emb_hbm.at[idx_v.at[pl.ds(j * CHUNK, CHUNK)]],
            bufs[j % 3],
            gsems[j % 3],
        )

    def scatter(j):
        buf = bufs[j % 3]
        a = pltpu.make_async_copy(
            buf.at[pl.ds(0, T), pl.ds(0, VOCAB)],
            out_hbm.at[base_b + BPC * j], ssems[j % 3])
        b = pltpu.make_async_copy(
            buf.at[pl.ds(T, T), pl.ds(0, VOCAB)],
            out_hbm.at[base_b + BPC * j + 1],
            ssems[j % 3])
        return a, b

    gathers = [gather(0), gather(1)]
    gathers[0].start()
    gathers[1].start()
    scatters = []
    for j in range(N_CHUNKS):
        if j + 2 < N_CHUNKS:
            if j >= 1:
                for cp in scatters[j - 1]:
                    cp.wait()
            g = gather(j + 2)
            g.start()
            gathers.append(g)
        gathers[j].wait()
        a, b = scatter(j)
        a.start()
        b.start()
        scatters.append((a, b))
    for j in (N_CHUNKS - 3, N_CHUNKS - 2, N_CHUNKS - 1):
        for cp in scatters[j]:
            cp.wait()


def kernel(xb, emb):
    idx = xb.reshape(-1)
    embp = jnp.pad(emb, ((0, 0), (0, 1024 - VOCAB)))
    return _gather_rows(embp, idx)
